# baseline (speedup 1.0000x reference)
import functools

import jax
import jax.numpy as jnp
from jax import lax
from jax.experimental import pallas as pl
from jax.experimental.pallas import tpu as pltpu
from jax.experimental.pallas import tpu_sc as plsc

N_EXPERTS = 64
CAPACITY_FACTOR = 1.25

N_TOKENS = 16384
TOP_K = 8
STREAM = N_TOKENS * TOP_K
NW = 16
CHUNK = STREAM // NW
NSTREAMS = 4
NSUB = 16 * NSTREAMS
SUB = CHUNK // NSUB
TOK_W = N_TOKENS // NW


def _sc_body(e_hbm, w_hbm, cap_hbm, wc_hbm, ov_hbm,
             e_v, w_v, pos_v, wc_v, cnt_v, off_v, tot_v, base_v, all_tot_v,
             cap_v, acc8_v, ov_v, w_sem, shared_tot, shared_wc):
    wid = lax.axis_index("s")
    iota = lax.iota(jnp.int32, 16)
    zeros16 = jnp.zeros((16,), jnp.int32)

    base_el = wid * CHUNK
    pltpu.sync_copy(e_hbm.at[pl.ds(base_el, CHUNK)], e_v)
    w_cp = pltpu.async_copy(w_hbm.at[pl.ds(base_el, CHUNK)], w_v, w_sem)
    pltpu.sync_copy(cap_hbm, cap_v)

    def zero_cnt(i, c):
        cnt_v[pl.ds(i * 16, 16)] = zeros16
        return c

    lax.fori_loop(0, (N_EXPERTS * NSUB) // 16, zero_cnt, 0)

    def scan_step(t, c):
        for s in range(NSTREAMS):
            idx = iota * SUB + (s * (16 * SUB) + t)
            e = plsc.load_gather(e_v, [idx])
            cidx = e * NSUB + (s * 16) + iota
            cnt = plsc.load_gather(cnt_v, [cidx])
            plsc.store_scatter(pos_v, [idx], cnt)
            plsc.store_scatter(cnt_v, [cidx], cnt + 1)
        return c

    lax.fori_loop(0, SUB, scan_step, 0)

    for g in range(N_EXPERTS // 16):
        def tot_step(j, acc, g=g):
            return acc + plsc.load_gather(cnt_v, [(g * 16 + iota) * NSUB + j])

        tot_v[pl.ds(g * 16, 16)] = lax.fori_loop(0, NSUB, tot_step, zeros16)

    pltpu.sync_copy(tot_v, shared_tot.at[wid])
    plsc.subcore_barrier()
    pltpu.sync_copy(shared_tot, all_tot_v)

    for g in range(N_EXPERTS // 16):
        def base_step(wp, acc, g=g):
            v = all_tot_v[wp, pl.ds(g * 16, 16)]
            return acc + v * (wp < wid).astype(jnp.int32)

        base_v[pl.ds(g * 16, 16)] = lax.fori_loop(0, NW, base_step, zeros16)

    def off_step(e, c):
        b = plsc.load_gather(base_v, [jnp.full((16,), 0, jnp.int32) + e])
        carry = b
        for k in range(NSTREAMS):
            vk = cnt_v[pl.ds(e * NSUB + k * 16, 16)]
            off_v[pl.ds(e * NSUB + k * 16, 16)] = plsc.cumsum(vk) - vk + carry
            carry = carry + jnp.sum(vk)
        return c

    lax.fori_loop(0, N_EXPERTS, off_step, 0)

    cap_vec = cap_v[...]
    w_cp.wait()

    def mask_step(t, c):
        sl = pl.ds(t * 16, 16)
        e = e_v[sl]
        j = t // (SUB // 16)
        off = plsc.load_gather(off_v, [e * NSUB + j])
        keep = pos_v[sl] + off < cap_vec
        wvec = w_v[sl]
        wc_v[sl] = jnp.where(keep, wvec, jnp.zeros_like(wvec))
        return c

    lax.fori_loop(0, CHUNK // 16, mask_step, 0)

    pltpu.sync_copy(wc_v, wc_hbm.at[pl.ds(base_el, CHUNK)])
    pltpu.sync_copy(wc_v, shared_wc.at[pl.ds(base_el, CHUNK)])
    plsc.subcore_barrier()

    for slot in range(TOP_K):
        pltpu.sync_copy(
            shared_wc.at[pl.ds(slot * N_TOKENS + wid * TOK_W, TOK_W)],
            acc8_v.at[slot])

    def ov_step(t, c):
        sl = pl.ds(t * 16, 16)
        s = acc8_v[0, sl]
        for slot in range(1, TOP_K):
            s = s + acc8_v[slot, sl]
        ov_v[sl] = (s == 0.0).astype(jnp.int32)
        return c

    lax.fori_loop(0, TOK_W // 16, ov_step, 0)

    pltpu.sync_copy(ov_v, ov_hbm.at[pl.ds(wid * TOK_W, TOK_W)])


@jax.jit
def _sc_call(e_flat, w_flat, cap16):
    mesh = plsc.VectorSubcoreMesh(
        core_axis_name="c", subcore_axis_name="s", num_cores=1, num_subcores=NW)
    return pl.kernel(
        _sc_body,
        out_type=[
            jax.ShapeDtypeStruct((STREAM,), jnp.float32),
            jax.ShapeDtypeStruct((N_TOKENS,), jnp.int32),
        ],
        mesh=mesh,
        compiler_params=pltpu.CompilerParams(needs_layout_passes=False),
        scratch_types=[
            pltpu.VMEM((CHUNK,), jnp.int32),
            pltpu.VMEM((CHUNK,), jnp.float32),
            pltpu.VMEM((CHUNK,), jnp.int32),
            pltpu.VMEM((CHUNK,), jnp.float32),
            pltpu.VMEM((N_EXPERTS * NSUB,), jnp.int32),
            pltpu.VMEM((N_EXPERTS * NSUB,), jnp.int32),
            pltpu.VMEM((N_EXPERTS,), jnp.int32),
            pltpu.VMEM((N_EXPERTS,), jnp.int32),
            pltpu.VMEM((NW, N_EXPERTS), jnp.int32),
            pltpu.VMEM((16,), jnp.int32),
            pltpu.VMEM((TOP_K, TOK_W), jnp.float32),
            pltpu.VMEM((TOK_W,), jnp.int32),
            pltpu.SemaphoreType.DMA,
            pltpu.VMEM_SHARED((NW, N_EXPERTS), jnp.int32),
            pltpu.VMEM_SHARED((STREAM,), jnp.float32),
        ],
    )(e_flat, w_flat, cap16)


def kernel(dispatch_weights, expert_indices, n_tokens):
    n, top_k = dispatch_weights.shape
    capacity = jnp.maximum(
        1, jnp.ceil(CAPACITY_FACTOR * n_tokens * top_k / N_EXPERTS)
    ).astype(jnp.int32)
    e_flat = expert_indices.T.reshape(-1).astype(jnp.int32)
    w_flat = dispatch_weights.T.reshape(-1)
    cap16 = jnp.full((16,), capacity, jnp.int32)
    wc_flat, ov = _sc_call(e_flat, w_flat, cap16)
    weights_capped = wc_flat.reshape(top_k, n).T
    overflow_mask = ov.astype(bool)
    return (weights_capped, expert_indices, overflow_mask)


# speedup vs baseline: 1.2880x; 1.2880x over previous
import jax
import jax.numpy as jnp
from jax import lax
from jax.experimental import pallas as pl
from jax.experimental.pallas import tpu as pltpu
from jax.experimental.pallas import tpu_sc as plsc

N_EXPERTS = 64
CAPACITY_FACTOR = 1.25

N_TOKENS = 16384
TOP_K = 8
STREAM = N_TOKENS * TOP_K
NW = 16
CHUNK = STREAM // NW
NSTREAMS = 4
NSUB = 16 * NSTREAMS
SUB = CHUNK // NSUB
SLICE = CHUNK // NSTREAMS
TOK_W = N_TOKENS // NW


def _sc_body(e_hbm, w_hbm, cap_hbm, wc_hbm, ov_hbm,
             e_v, w_v, pos_v, wc_v, cnt_all, off_v, tot_v, base_v, all_tot_v,
             cap_v, acc8_v, ov_v, w_sem, shared_tot, shared_wc):
    wid = lax.axis_index("s")
    iota = lax.iota(jnp.int32, 16)
    zeros16 = jnp.zeros((16,), jnp.int32)

    base_el = wid * CHUNK
    pltpu.sync_copy(e_hbm.at[pl.ds(base_el, CHUNK)], e_v)
    w_cp = pltpu.async_copy(w_hbm.at[pl.ds(base_el, CHUNK)], w_v, w_sem)
    pltpu.sync_copy(cap_hbm, cap_v)

    @plsc.parallel_loop(0, NSTREAMS * N_EXPERTS * 16, step=16)
    def _zero(i):
        cnt_all[pl.ds(i, 16)] = zeros16

    def scan_step(t, c):
        @plsc.parallel_loop(0, NSTREAMS, unroll=NSTREAMS)
        def _streams(k):
            idx = iota * SUB + (k * SLICE + t)
            e = plsc.load_gather(e_v, [idx])
            cidx = k * (N_EXPERTS * 16) + e * 16 + iota
            cnt = plsc.load_gather(cnt_all, [cidx])
            plsc.store_scatter(pos_v, [idx], cnt)
            plsc.store_scatter(cnt_all, [cidx], cnt + 1)

        return c

    lax.fori_loop(0, SUB, scan_step, 0)

    for g in range(N_EXPERTS // 16):
        acc = zeros16
        for k in range(NSTREAMS):
            def tot_step(j, a, g=g, k=k):
                return a + plsc.load_gather(
                    cnt_all, [k * (N_EXPERTS * 16) + (g * 16 + iota) * 16 + j])

            acc = lax.fori_loop(0, 16, tot_step, acc)
        tot_v[pl.ds(g * 16, 16)] = acc

    pltpu.sync_copy(tot_v, shared_tot.at[wid])
    plsc.subcore_barrier()
    pltpu.sync_copy(shared_tot, all_tot_v)

    for g in range(N_EXPERTS // 16):
        def base_step(wp, acc, g=g):
            v = all_tot_v[wp, pl.ds(g * 16, 16)]
            return acc + v * (wp < wid).astype(jnp.int32)

        base_v[pl.ds(g * 16, 16)] = lax.fori_loop(0, NW, base_step, zeros16)

    def off_step(e, c):
        carry = plsc.load_gather(base_v, [jnp.full((16,), 0, jnp.int32) + e])
        for k in range(NSTREAMS):
            vk = cnt_all[pl.ds(k * (N_EXPERTS * 16) + e * 16, 16)]
            off_v[pl.ds(e * NSUB + k * 16, 16)] = plsc.cumsum(vk) - vk + carry
            carry = carry + jnp.sum(vk)
        return c

    lax.fori_loop(0, N_EXPERTS, off_step, 0)

    cap_vec = cap_v[...]
    w_cp.wait()

    @plsc.parallel_loop(0, CHUNK, step=16, unroll=2)
    def _mask(i):
        sl = pl.ds(i, 16)
        e = e_v[sl]
        u = i // SUB
        off = plsc.load_gather(off_v, [e * NSUB + u])
        keep = pos_v[sl] + off < cap_vec
        wvec = w_v[sl]
        wc_v[sl] = jnp.where(keep, wvec, jnp.zeros_like(wvec))

    pltpu.sync_copy(wc_v, wc_hbm.at[pl.ds(base_el, CHUNK)])
    pltpu.sync_copy(wc_v, shared_wc.at[pl.ds(base_el, CHUNK)])
    plsc.subcore_barrier()

    for slot in range(TOP_K):
        pltpu.sync_copy(
            shared_wc.at[pl.ds(slot * N_TOKENS + wid * TOK_W, TOK_W)],
            acc8_v.at[slot])

    @plsc.parallel_loop(0, TOK_W, step=16, unroll=2)
    def _ov(i):
        sl = pl.ds(i, 16)
        s = acc8_v[0, sl]
        for slot in range(1, TOP_K):
            s = s + acc8_v[slot, sl]
        ov_v[sl] = (s == 0.0).astype(jnp.int32)

    pltpu.sync_copy(ov_v, ov_hbm.at[pl.ds(wid * TOK_W, TOK_W)])


@jax.jit
def _sc_call(e_flat, w_flat, cap16):
    mesh = plsc.VectorSubcoreMesh(
        core_axis_name="c", subcore_axis_name="s", num_cores=1, num_subcores=NW)
    return pl.kernel(
        _sc_body,
        out_type=[
            jax.ShapeDtypeStruct((STREAM,), jnp.float32),
            jax.ShapeDtypeStruct((N_TOKENS,), jnp.int32),
        ],
        mesh=mesh,
        compiler_params=pltpu.CompilerParams(needs_layout_passes=False),
        scratch_types=[
            pltpu.VMEM((CHUNK,), jnp.int32),
            pltpu.VMEM((CHUNK,), jnp.float32),
            pltpu.VMEM((CHUNK,), jnp.int32),
            pltpu.VMEM((CHUNK,), jnp.float32),
            pltpu.VMEM((NSTREAMS * N_EXPERTS * 16,), jnp.int32),
            pltpu.VMEM((N_EXPERTS * NSUB,), jnp.int32),
            pltpu.VMEM((N_EXPERTS,), jnp.int32),
            pltpu.VMEM((N_EXPERTS,), jnp.int32),
            pltpu.VMEM((NW, N_EXPERTS), jnp.int32),
            pltpu.VMEM((16,), jnp.int32),
            pltpu.VMEM((TOP_K, TOK_W), jnp.float32),
            pltpu.VMEM((TOK_W,), jnp.int32),
            pltpu.SemaphoreType.DMA,
            pltpu.VMEM_SHARED((NW, N_EXPERTS), jnp.int32),
            pltpu.VMEM_SHARED((STREAM,), jnp.float32),
        ],
    )(e_flat, w_flat, cap16)


def kernel(dispatch_weights, expert_indices, n_tokens):
    n, top_k = dispatch_weights.shape
    capacity = jnp.maximum(
        1, jnp.ceil(CAPACITY_FACTOR * n_tokens * top_k / N_EXPERTS)
    ).astype(jnp.int32)
    e_flat = expert_indices.T.reshape(-1).astype(jnp.int32)
    w_flat = dispatch_weights.T.reshape(-1)
    cap16 = jnp.full((16,), capacity, jnp.int32)
    wc_flat, ov = _sc_call(e_flat, w_flat, cap16)
    weights_capped = wc_flat.reshape(top_k, n).T
    overflow_mask = ov.astype(bool)
    return (weights_capped, expert_indices, overflow_mask)
